# Initial kernel scaffold; baseline (speedup 1.0000x reference)
import jax
import jax.numpy as jnp
from jax.experimental import pallas as pl


def kernel(x, edge_index, W1, b1, W2, b2, Wl, bl):
    raise NotImplementedError("write your pallas kernel here")


# speedup vs baseline: 13.6938x; 13.6938x over previous
import functools

import jax
import jax.numpy as jnp
from jax import lax
from jax.experimental import pallas as pl
from jax.experimental.pallas import tpu as pltpu
from jax.experimental.pallas import tpu_sc as plsc

N = 10000
E = 320000
D_IN = 128
DH = 64

NC = 2
NS = 16
NW = NC * NS

N2 = 10240
RS = N2 // NS
BLK_E = 128
NBLK = 80
E_PAD = NW * NBLK * BLK_E
EC = E // NW

BN = 2048

_mesh = plsc.VectorSubcoreMesh(
    core_axis_name="c", subcore_axis_name="s", num_cores=NC, num_subcores=NS
)


DW = 16


def _hist_body(dst_hbm, deg_hbm, didx_v, ones_v, dacc_sh):
    c = lax.axis_index("c")
    s = lax.axis_index("s")
    wid = s * NC + c
    pltpu.sync_copy(dst_hbm.at[wid], didx_v)
    zeros16 = jnp.zeros((16,), jnp.float32)

    def zrow(r, carry):
        ones_v[r, pl.ds(0, DW)] = zeros16
        return carry

    lax.fori_loop(0, BLK_E, zrow, 0)
    base = s * RS
    for k in range(RS // BLK_E):
        pltpu.sync_copy(ones_v, dacc_sh.at[pl.ds(base + k * BLK_E, BLK_E)])
    ones16 = jnp.ones((16,), jnp.float32)

    def orow(r, carry):
        ones_v[r, pl.ds(0, DW)] = ones16
        return carry

    lax.fori_loop(0, BLK_E, orow, 0)
    plsc.subcore_barrier()

    def eblk(j, carry):
        pltpu.sync_copy(ones_v, dacc_sh.at[didx_v.at[j]], add=True)
        return carry

    lax.fori_loop(0, NBLK, eblk, 0)
    plsc.subcore_barrier()
    pltpu.sync_copy(dacc_sh.at[pl.ds(base, RS)], deg_hbm.at[c, pl.ds(base, RS)])


_hist = pl.kernel(
    _hist_body,
    jax.ShapeDtypeStruct((NC, N2, DW), jnp.float32),
    mesh=_mesh,
    compiler_params=pltpu.CompilerParams(use_tc_tiling_on_sc=False),
    scratch_types=[
        pltpu.VMEM((NBLK, BLK_E), jnp.int32),
        pltpu.VMEM((BLK_E, DW), jnp.float32),
        pltpu.VMEM_SHARED((N2, DW), jnp.float32),
    ],
)


def _agg_body(g_hbm, src_hbm, dst_hbm, out_hbm, sidx_v, didx_v, msg_v, acc_sh, sem):
    c = lax.axis_index("c")
    s = lax.axis_index("s")
    wid = s * NC + c
    pltpu.sync_copy(src_hbm.at[wid], sidx_v)
    pltpu.sync_copy(dst_hbm.at[wid], didx_v)

    zeros16 = jnp.zeros((16,), jnp.float32)

    def zrow(r, carry):
        for q in range(DH // 16):
            msg_v[r, pl.ds(q * 16, 16)] = zeros16
        return carry

    lax.fori_loop(0, BLK_E, zrow, 0)
    base = s * RS
    for k in range(RS // BLK_E):
        pltpu.sync_copy(msg_v, acc_sh.at[pl.ds(base + k * BLK_E, BLK_E)])
    plsc.subcore_barrier()

    def eblk(j, carry):
        pltpu.async_copy(g_hbm.at[sidx_v.at[j]], msg_v, sem).wait()
        pltpu.sync_copy(msg_v, acc_sh.at[didx_v.at[j]], add=True)
        return carry

    lax.fori_loop(0, NBLK, eblk, 0)
    plsc.subcore_barrier()
    pltpu.sync_copy(acc_sh.at[pl.ds(base, RS)], out_hbm.at[c, pl.ds(base, RS)])


_agg = pl.kernel(
    _agg_body,
    jax.ShapeDtypeStruct((NC, N2, DH), jnp.float32),
    mesh=_mesh,
    compiler_params=pltpu.CompilerParams(use_tc_tiling_on_sc=False),
    scratch_types=[
        pltpu.VMEM((NBLK, BLK_E), jnp.int32),
        pltpu.VMEM((NBLK, BLK_E), jnp.int32),
        pltpu.VMEM((BLK_E, DH), jnp.float32),
        pltpu.VMEM_SHARED((N2, DH), jnp.float32),
        pltpu.SemaphoreType.DMA,
    ],
)


def _dinv_of(deg_blk):
    return lax.rsqrt(deg_blk[0, :, :1] + deg_blk[1, :, :1] + 1.0)


def _tc1_body(deg_ref, x_ref, w1_ref, g1_ref):
    dinv = _dinv_of(deg_ref[...])
    h = jnp.dot(x_ref[...], w1_ref[...], preferred_element_type=jnp.float32)
    g1_ref[...] = h * dinv


def _tc2_body(acc_ref, g1_ref, deg_ref, w2_ref, b1_ref, g2_ref):
    dinv = _dinv_of(deg_ref[...])
    a = acc_ref[0] + acc_ref[1] + g1_ref[...]
    h = jnp.maximum(a * dinv + b1_ref[...], 0.0)
    h2 = jnp.dot(h, w2_ref[...], preferred_element_type=jnp.float32)
    g2_ref[...] = h2 * dinv


def _tc3_body(acc_ref, g2_ref, deg_ref, wl_ref, b2_ref, bl_ref, y_ref):
    dinv = _dinv_of(deg_ref[...])
    a = acc_ref[0] + acc_ref[1] + g2_ref[...]
    h = jnp.maximum(a * dinv + b2_ref[...], 0.0)
    y_ref[...] = jnp.sum(h * wl_ref[...], axis=1) + bl_ref[0, 0]


_deg_spec = pl.BlockSpec((NC, BN, DW), lambda i: (0, i, 0))
_row_spec = pl.BlockSpec((BN, DH), lambda i: (i, 0))
_acc_spec = pl.BlockSpec((NC, BN, DH), lambda i: (0, i, 0))
_vec_spec = pl.BlockSpec((1, DH), lambda i: (0, 0))

_tc1 = pl.pallas_call(
    _tc1_body,
    grid=(N2 // BN,),
    in_specs=[
        _deg_spec,
        pl.BlockSpec((BN, D_IN), lambda i: (i, 0)),
        pl.BlockSpec((D_IN, DH), lambda i: (0, 0)),
    ],
    out_specs=_row_spec,
    out_shape=jax.ShapeDtypeStruct((N2, DH), jnp.float32),
)

_tc2 = pl.pallas_call(
    _tc2_body,
    grid=(N2 // BN,),
    in_specs=[
        _acc_spec,
        _row_spec,
        _deg_spec,
        pl.BlockSpec((DH, DH), lambda i: (0, 0)),
        _vec_spec,
    ],
    out_specs=_row_spec,
    out_shape=jax.ShapeDtypeStruct((N2, DH), jnp.float32),
)

_tc3 = pl.pallas_call(
    _tc3_body,
    grid=(N2 // BN,),
    in_specs=[
        _acc_spec,
        _row_spec,
        _deg_spec,
        _vec_spec,
        _vec_spec,
        _vec_spec,
    ],
    out_specs=pl.BlockSpec((BN,), lambda i: (i,)),
    out_shape=jax.ShapeDtypeStruct((N2,), jnp.float32),
)


def kernel(x, edge_index, W1, b1, W2, b2, Wl, bl):
    src = edge_index[0]
    dst = edge_index[1]
    pad = jnp.full((E_PAD - E,), N, dtype=edge_index.dtype)
    src_r = jnp.concatenate([src, pad]).reshape(NW, NBLK, BLK_E)
    dst_r = jnp.concatenate([dst, pad]).reshape(NW, NBLK, BLK_E)
    x_pad = jnp.pad(x, ((0, N2 - N), (0, 0)))

    deg_parts = _hist(dst_r)
    g1 = _tc1(deg_parts, x_pad, W1)
    acc1 = _agg(g1, src_r, dst_r)
    g2 = _tc2(acc1, g1, deg_parts, W2, b1.reshape(1, DH))
    acc2 = _agg(g2, src_r, dst_r)
    y = _tc3(
        acc2,
        g2,
        deg_parts,
        Wl.reshape(1, DH),
        b2.reshape(1, DH),
        jnp.broadcast_to(bl.reshape(1, 1), (1, DH)),
    )
    return y[:N]


# speedup vs baseline: 15.7667x; 1.1514x over previous
import functools

import jax
import jax.numpy as jnp
from jax import lax
from jax.experimental import pallas as pl
from jax.experimental.pallas import tpu as pltpu
from jax.experimental.pallas import tpu_sc as plsc

N = 10000
E = 320000
D_IN = 128
DH = 64

NC = 2
NS = 16
NW = NC * NS

N2 = 10240
RS = N2 // NS
BLK_E = 128
NBLK = 80
E_PAD = NW * NBLK * BLK_E
EC = E // NW

BN = 2048

_mesh = plsc.VectorSubcoreMesh(
    core_axis_name="c", subcore_axis_name="s", num_cores=NC, num_subcores=NS
)


DW = 16


def _hist_body(dst_hbm, deg_hbm, didx_v, ones_v, dacc_sh):
    c = lax.axis_index("c")
    s = lax.axis_index("s")
    wid = s * NC + c
    pltpu.sync_copy(dst_hbm.at[wid], didx_v)
    zeros16 = jnp.zeros((16,), jnp.float32)

    def zrow(r, carry):
        ones_v[r, pl.ds(0, DW)] = zeros16
        return carry

    lax.fori_loop(0, BLK_E, zrow, 0)
    base = s * RS
    for k in range(RS // BLK_E):
        pltpu.sync_copy(ones_v, dacc_sh.at[pl.ds(base + k * BLK_E, BLK_E)])
    ones16 = jnp.ones((16,), jnp.float32)

    def orow(r, carry):
        ones_v[r, pl.ds(0, DW)] = ones16
        return carry

    lax.fori_loop(0, BLK_E, orow, 0)
    plsc.subcore_barrier()

    def eblk(j, carry):
        pltpu.sync_copy(ones_v, dacc_sh.at[didx_v.at[j]], add=True)
        return carry

    lax.fori_loop(0, NBLK, eblk, 0)
    plsc.subcore_barrier()
    pltpu.sync_copy(dacc_sh.at[pl.ds(base, RS)], deg_hbm.at[c, pl.ds(base, RS)])


_hist = pl.kernel(
    _hist_body,
    jax.ShapeDtypeStruct((NC, N2, DW), jnp.float32),
    mesh=_mesh,
    compiler_params=pltpu.CompilerParams(use_tc_tiling_on_sc=False),
    scratch_types=[
        pltpu.VMEM((NBLK, BLK_E), jnp.int32),
        pltpu.VMEM((BLK_E, DW), jnp.float32),
        pltpu.VMEM_SHARED((N2, DW), jnp.float32),
    ],
)


NBUF = 4
NGRP = NBLK // (2 * NBUF)


def _agg_body(g_hbm, src_hbm, dst_hbm, out_hbm, sidx_v, didx_v, msg_a, msg_b,
              acc_sh, gsem_a, ssem_a, gsem_b, ssem_b):
    c = lax.axis_index("c")
    s = lax.axis_index("s")
    wid = s * NC + c
    pltpu.sync_copy(src_hbm.at[wid], sidx_v)
    pltpu.sync_copy(dst_hbm.at[wid], didx_v)

    zeros16 = jnp.zeros((16,), jnp.float32)

    def zrow(r, carry):
        for q in range(DH // 16):
            msg_a[0, r, pl.ds(q * 16, 16)] = zeros16
        return carry

    lax.fori_loop(0, BLK_E, zrow, 0)
    base = s * RS
    for k in range(RS // BLK_E):
        pltpu.sync_copy(msg_a.at[0], acc_sh.at[pl.ds(base + k * BLK_E, BLK_E)])
    plsc.subcore_barrier()

    def _gather(j, buf_ref, sem):
        pltpu.async_copy(g_hbm.at[sidx_v.at[j]], buf_ref, sem)

    def _gather_wait(j, buf_ref, sem):
        pltpu.make_async_copy(g_hbm.at[sidx_v.at[j]], buf_ref, sem).wait()

    def _scat(j, buf_ref, sem):
        pltpu.async_copy(buf_ref, acc_sh.at[didx_v.at[j]], sem, add=True)

    def _scat_wait(j, buf_ref, sem):
        pltpu.make_async_copy(buf_ref, acc_sh.at[didx_v.at[j]], sem).wait()

    for b in range(NBUF):
        _gather(b, msg_a.at[b], gsem_a)

    def grp(gi, carry):
        ja = gi * 2 * NBUF
        jb = ja + NBUF
        for b in range(NBUF):
            _gather(jb + b, msg_b.at[b], gsem_b)
        for b in range(NBUF):
            _gather_wait(ja + b, msg_a.at[b], gsem_a)
        for b in range(NBUF):
            _scat(ja + b, msg_a.at[b], ssem_a)
        for b in range(NBUF):
            _scat_wait(ja + b, msg_a.at[b], ssem_a)

        @pl.when(gi < NGRP - 1)
        def _refill_a():
            for b in range(NBUF):
                _gather(ja + 2 * NBUF + b, msg_a.at[b], gsem_a)

        for b in range(NBUF):
            _gather_wait(jb + b, msg_b.at[b], gsem_b)
        for b in range(NBUF):
            _scat(jb + b, msg_b.at[b], ssem_b)
        for b in range(NBUF):
            _scat_wait(jb + b, msg_b.at[b], ssem_b)
        return carry

    lax.fori_loop(0, NGRP, grp, 0)
    plsc.subcore_barrier()
    pltpu.sync_copy(acc_sh.at[pl.ds(base, RS)], out_hbm.at[c, pl.ds(base, RS)])


_agg = pl.kernel(
    _agg_body,
    jax.ShapeDtypeStruct((NC, N2, DH), jnp.float32),
    mesh=_mesh,
    compiler_params=pltpu.CompilerParams(use_tc_tiling_on_sc=False),
    scratch_types=[
        pltpu.VMEM((NBLK, BLK_E), jnp.int32),
        pltpu.VMEM((NBLK, BLK_E), jnp.int32),
        pltpu.VMEM((NBUF, BLK_E, DH), jnp.float32),
        pltpu.VMEM((NBUF, BLK_E, DH), jnp.float32),
        pltpu.VMEM_SHARED((N2, DH), jnp.float32),
        pltpu.SemaphoreType.DMA,
        pltpu.SemaphoreType.DMA,
        pltpu.SemaphoreType.DMA,
        pltpu.SemaphoreType.DMA,
    ],
)


def _dinv_of(deg_blk):
    return lax.rsqrt(deg_blk[0, :, :1] + deg_blk[1, :, :1] + 1.0)


def _tc1_body(deg_ref, x_ref, w1_ref, g1_ref):
    dinv = _dinv_of(deg_ref[...])
    h = jnp.dot(x_ref[...], w1_ref[...], preferred_element_type=jnp.float32)
    g1_ref[...] = h * dinv


def _tc2_body(acc_ref, g1_ref, deg_ref, w2_ref, b1_ref, g2_ref):
    dinv = _dinv_of(deg_ref[...])
    a = acc_ref[0] + acc_ref[1] + g1_ref[...]
    h = jnp.maximum(a * dinv + b1_ref[...], 0.0)
    h2 = jnp.dot(h, w2_ref[...], preferred_element_type=jnp.float32)
    g2_ref[...] = h2 * dinv


def _tc3_body(acc_ref, g2_ref, deg_ref, wl_ref, b2_ref, bl_ref, y_ref):
    dinv = _dinv_of(deg_ref[...])
    a = acc_ref[0] + acc_ref[1] + g2_ref[...]
    h = jnp.maximum(a * dinv + b2_ref[...], 0.0)
    y_ref[...] = jnp.sum(h * wl_ref[...], axis=1) + bl_ref[0, 0]


_deg_spec = pl.BlockSpec((NC, BN, DW), lambda i: (0, i, 0))
_row_spec = pl.BlockSpec((BN, DH), lambda i: (i, 0))
_acc_spec = pl.BlockSpec((NC, BN, DH), lambda i: (0, i, 0))
_vec_spec = pl.BlockSpec((1, DH), lambda i: (0, 0))

_tc1 = pl.pallas_call(
    _tc1_body,
    grid=(N2 // BN,),
    in_specs=[
        _deg_spec,
        pl.BlockSpec((BN, D_IN), lambda i: (i, 0)),
        pl.BlockSpec((D_IN, DH), lambda i: (0, 0)),
    ],
    out_specs=_row_spec,
    out_shape=jax.ShapeDtypeStruct((N2, DH), jnp.float32),
)

_tc2 = pl.pallas_call(
    _tc2_body,
    grid=(N2 // BN,),
    in_specs=[
        _acc_spec,
        _row_spec,
        _deg_spec,
        pl.BlockSpec((DH, DH), lambda i: (0, 0)),
        _vec_spec,
    ],
    out_specs=_row_spec,
    out_shape=jax.ShapeDtypeStruct((N2, DH), jnp.float32),
)

_tc3 = pl.pallas_call(
    _tc3_body,
    grid=(N2 // BN,),
    in_specs=[
        _acc_spec,
        _row_spec,
        _deg_spec,
        _vec_spec,
        _vec_spec,
        _vec_spec,
    ],
    out_specs=pl.BlockSpec((BN,), lambda i: (i,)),
    out_shape=jax.ShapeDtypeStruct((N2,), jnp.float32),
)


def kernel(x, edge_index, W1, b1, W2, b2, Wl, bl):
    src = edge_index[0]
    dst = edge_index[1]
    pad = jnp.full((E_PAD - E,), N, dtype=edge_index.dtype)
    src_r = jnp.concatenate([src, pad]).reshape(NW, NBLK, BLK_E)
    dst_r = jnp.concatenate([dst, pad]).reshape(NW, NBLK, BLK_E)
    x_pad = jnp.pad(x, ((0, N2 - N), (0, 0)))

    deg_parts = _hist(dst_r)
    g1 = _tc1(deg_parts, x_pad, W1)
    acc1 = _agg(g1, src_r, dst_r)
    g2 = _tc2(acc1, g1, deg_parts, W2, b1.reshape(1, DH))
    acc2 = _agg(g2, src_r, dst_r)
    y = _tc3(
        acc2,
        g2,
        deg_parts,
        Wl.reshape(1, DH),
        b2.reshape(1, DH),
        jnp.broadcast_to(bl.reshape(1, 1), (1, DH)),
    )
    return y[:N]


# speedup vs baseline: 15.7762x; 1.0006x over previous
import functools

import jax
import jax.numpy as jnp
from jax import lax
from jax.experimental import pallas as pl
from jax.experimental.pallas import tpu as pltpu
from jax.experimental.pallas import tpu_sc as plsc

N = 10000
E = 320000
D_IN = 128
DH = 64

NC = 2
NS = 16
NW = NC * NS

N2 = 10240
RS = N2 // NS
BLK_E = 128
NBLK = 80
E_PAD = NW * NBLK * BLK_E
EC = E // NW

BN = 2048

_mesh = plsc.VectorSubcoreMesh(
    core_axis_name="c", subcore_axis_name="s", num_cores=NC, num_subcores=NS
)


DW = 16


def _hist_body(dst_hbm, deg_hbm, didx_v, ones_v, dacc_sh):
    c = lax.axis_index("c")
    s = lax.axis_index("s")
    wid = s * NC + c
    pltpu.sync_copy(dst_hbm.at[wid], didx_v)
    zeros16 = jnp.zeros((16,), jnp.float32)

    def zrow(r, carry):
        ones_v[r, pl.ds(0, DW)] = zeros16
        return carry

    lax.fori_loop(0, BLK_E, zrow, 0)
    base = s * RS
    for k in range(RS // BLK_E):
        pltpu.sync_copy(ones_v, dacc_sh.at[pl.ds(base + k * BLK_E, BLK_E)])
    ones16 = jnp.ones((16,), jnp.float32)

    def orow(r, carry):
        ones_v[r, pl.ds(0, DW)] = ones16
        return carry

    lax.fori_loop(0, BLK_E, orow, 0)
    plsc.subcore_barrier()

    def eblk(j, carry):
        pltpu.sync_copy(ones_v, dacc_sh.at[didx_v.at[j]], add=True)
        return carry

    lax.fori_loop(0, NBLK, eblk, 0)
    plsc.subcore_barrier()
    pltpu.sync_copy(dacc_sh.at[pl.ds(base, RS)], deg_hbm.at[c, pl.ds(base, RS)])


_hist = pl.kernel(
    _hist_body,
    jax.ShapeDtypeStruct((NC, N2, DW), jnp.float32),
    mesh=_mesh,
    compiler_params=pltpu.CompilerParams(use_tc_tiling_on_sc=False),
    scratch_types=[
        pltpu.VMEM((NBLK, BLK_E), jnp.int32),
        pltpu.VMEM((BLK_E, DW), jnp.float32),
        pltpu.VMEM_SHARED((N2, DW), jnp.float32),
    ],
)


NBUF = 4
NGRP = NBLK // (2 * NBUF)


def _agg_body(g_hbm, src_hbm, dst_hbm, out_hbm, sidx_v, didx_v, msg_a, msg_b,
              acc_sh, gsem_a, ssem_a, gsem_b, ssem_b):
    c = lax.axis_index("c")
    s = lax.axis_index("s")
    wid = s * NC + c
    pltpu.sync_copy(src_hbm.at[wid], sidx_v)
    pltpu.sync_copy(dst_hbm.at[wid], didx_v)
    base = s * RS

    zeros16 = jnp.zeros((16,), jnp.float32)

    def zrow(r, carry):
        for q in range(DH // 16):
            msg_a[0, r, pl.ds(q * 16, 16)] = zeros16
        return carry

    lax.fori_loop(0, BLK_E, zrow, 0)
    for k in range(RS // BLK_E):
        pltpu.sync_copy(msg_a.at[0], acc_sh.at[pl.ds(base + k * BLK_E, BLK_E)])
    plsc.subcore_barrier()

    def _gather(j, buf_ref, sem):
        pltpu.async_copy(g_hbm.at[sidx_v.at[j]], buf_ref, sem)

    def _gather_wait(j, buf_ref, sem):
        pltpu.make_async_copy(g_hbm.at[sidx_v.at[j]], buf_ref, sem).wait()

    def _scat(j, buf_ref, sem):
        pltpu.async_copy(buf_ref, acc_sh.at[didx_v.at[j]], sem, add=True)

    def _scat_wait(j, buf_ref, sem):
        pltpu.make_async_copy(buf_ref, acc_sh.at[didx_v.at[j]], sem).wait()

    for b in range(NBUF):
        _gather(b, msg_a.at[b], gsem_a)

    def grp(gi, carry):
        ja = gi * 2 * NBUF
        jb = ja + NBUF
        for b in range(NBUF):
            _gather(jb + b, msg_b.at[b], gsem_b)
        for b in range(NBUF):
            _gather_wait(ja + b, msg_a.at[b], gsem_a)
        for b in range(NBUF):
            _scat(ja + b, msg_a.at[b], ssem_a)
        for b in range(NBUF):
            _scat_wait(ja + b, msg_a.at[b], ssem_a)

        @pl.when(gi < NGRP - 1)
        def _refill_a():
            for b in range(NBUF):
                _gather(ja + 2 * NBUF + b, msg_a.at[b], gsem_a)

        for b in range(NBUF):
            _gather_wait(jb + b, msg_b.at[b], gsem_b)
        for b in range(NBUF):
            _scat(jb + b, msg_b.at[b], ssem_b)
        for b in range(NBUF):
            _scat_wait(jb + b, msg_b.at[b], ssem_b)
        return carry

    lax.fori_loop(0, NGRP, grp, 0)
    plsc.subcore_barrier()
    pltpu.sync_copy(acc_sh.at[pl.ds(base, RS)], out_hbm.at[c].at[pl.ds(base, RS)])


_agg = pl.kernel(
    _agg_body,
    jax.ShapeDtypeStruct((NC, N2, DH), jnp.float32),
    mesh=_mesh,
    compiler_params=pltpu.CompilerParams(use_tc_tiling_on_sc=False),
    scratch_types=[
        pltpu.VMEM((NBLK, BLK_E), jnp.int32),
        pltpu.VMEM((NBLK, BLK_E), jnp.int32),
        pltpu.VMEM((NBUF, BLK_E, DH), jnp.float32),
        pltpu.VMEM((NBUF, BLK_E, DH), jnp.float32),
        pltpu.VMEM_SHARED((N2, DH), jnp.float32),
        pltpu.SemaphoreType.DMA,
        pltpu.SemaphoreType.DMA,
        pltpu.SemaphoreType.DMA,
        pltpu.SemaphoreType.DMA,
    ],
)


def _dinv_of(deg_blk):
    return lax.rsqrt(deg_blk[0, :, :1] + deg_blk[1, :, :1] + 1.0)


def _tc1_body(deg_ref, x_ref, w1_ref, g1_ref):
    dinv = _dinv_of(deg_ref[...])
    h = jnp.dot(x_ref[...], w1_ref[...], preferred_element_type=jnp.float32)
    g1_ref[...] = h * dinv


def _tc2_body(acc_ref, g1_ref, deg_ref, w2_ref, b1_ref, g2_ref):
    dinv = _dinv_of(deg_ref[...])
    a = acc_ref[0] + acc_ref[1] + g1_ref[...]
    h = jnp.maximum(a * dinv + b1_ref[...], 0.0)
    h2 = jnp.dot(h, w2_ref[...], preferred_element_type=jnp.float32)
    g2_ref[...] = h2 * dinv


def _tc3_body(acc_ref, g2_ref, deg_ref, wl_ref, b2_ref, bl_ref, y_ref):
    dinv = _dinv_of(deg_ref[...])
    a = acc_ref[0] + acc_ref[1] + g2_ref[...]
    h = jnp.maximum(a * dinv + b2_ref[...], 0.0)
    y_ref[...] = jnp.sum(h * wl_ref[...], axis=1) + bl_ref[0, 0]


_deg_spec = pl.BlockSpec((NC, BN, DW), lambda i: (0, i, 0))
_row_spec = pl.BlockSpec((BN, DH), lambda i: (i, 0))
_acc_spec = pl.BlockSpec((NC, BN, DH), lambda i: (0, i, 0))
_vec_spec = pl.BlockSpec((1, DH), lambda i: (0, 0))

_tc1 = pl.pallas_call(
    _tc1_body,
    grid=(N2 // BN,),
    in_specs=[
        _deg_spec,
        pl.BlockSpec((BN, D_IN), lambda i: (i, 0)),
        pl.BlockSpec((D_IN, DH), lambda i: (0, 0)),
    ],
    out_specs=_row_spec,
    out_shape=jax.ShapeDtypeStruct((N2, DH), jnp.float32),
)

_tc2 = pl.pallas_call(
    _tc2_body,
    grid=(N2 // BN,),
    in_specs=[
        _acc_spec,
        _row_spec,
        _deg_spec,
        pl.BlockSpec((DH, DH), lambda i: (0, 0)),
        _vec_spec,
    ],
    out_specs=_row_spec,
    out_shape=jax.ShapeDtypeStruct((N2, DH), jnp.float32),
)

_tc3 = pl.pallas_call(
    _tc3_body,
    grid=(N2 // BN,),
    in_specs=[
        _acc_spec,
        _row_spec,
        _deg_spec,
        _vec_spec,
        _vec_spec,
        _vec_spec,
    ],
    out_specs=pl.BlockSpec((BN,), lambda i: (i,)),
    out_shape=jax.ShapeDtypeStruct((N2,), jnp.float32),
)


def kernel(x, edge_index, W1, b1, W2, b2, Wl, bl):
    src = edge_index[0]
    dst = edge_index[1]
    pad = jnp.full((E_PAD - E,), N, dtype=edge_index.dtype)
    src_r = jnp.concatenate([src, pad]).reshape(NW, NBLK, BLK_E)
    dst_r = jnp.concatenate([dst, pad]).reshape(NW, NBLK, BLK_E)
    x_pad = jnp.pad(x, ((0, N2 - N), (0, 0)))

    deg_parts = _hist(dst_r)
    g1 = _tc1(deg_parts, x_pad, W1)
    acc1 = _agg(g1, src_r, dst_r)
    g2 = _tc2(acc1, g1, deg_parts, W2, b1.reshape(1, DH))
    acc2 = _agg(g2, src_r, dst_r)
    y = _tc3(
        acc2,
        g2,
        deg_parts,
        Wl.reshape(1, DH),
        b2.reshape(1, DH),
        jnp.broadcast_to(bl.reshape(1, 1), (1, DH)),
    )
    return y[:N]


# speedup vs baseline: 43.0459x; 2.7285x over previous
import functools

import jax
import jax.numpy as jnp
from jax import lax
from jax.experimental import pallas as pl
from jax.experimental.pallas import tpu as pltpu
from jax.experimental.pallas import tpu_sc as plsc

N = 10000
E = 320000
D_IN = 128
DH = 64

NC = 2
NS = 16
NW = NC * NS

N2 = 10240
RS = N2 // NS
BLK_E = 128
NBLK = 80
E_PAD = NW * NBLK * BLK_E
EC = E // NW

BN = 2048

_mesh = plsc.VectorSubcoreMesh(
    core_axis_name="c", subcore_axis_name="s", num_cores=NC, num_subcores=NS
)


DW = 16


def _hist_body(dst_hbm, deg_hbm, didx_v, ones_v, dacc_sh):
    c = lax.axis_index("c")
    s = lax.axis_index("s")
    wid = s * NC + c
    pltpu.sync_copy(dst_hbm.at[wid], didx_v)
    zeros16 = jnp.zeros((16,), jnp.float32)

    def zrow(r, carry):
        ones_v[r, pl.ds(0, DW)] = zeros16
        return carry

    lax.fori_loop(0, BLK_E, zrow, 0)
    base = s * RS
    for k in range(RS // BLK_E):
        pltpu.sync_copy(ones_v, dacc_sh.at[pl.ds(base + k * BLK_E, BLK_E)])
    ones16 = jnp.ones((16,), jnp.float32)

    def orow(r, carry):
        ones_v[r, pl.ds(0, DW)] = ones16
        return carry

    lax.fori_loop(0, BLK_E, orow, 0)
    plsc.subcore_barrier()

    def eblk(j, carry):
        pltpu.sync_copy(ones_v, dacc_sh.at[didx_v.at[j]], add=True)
        return carry

    lax.fori_loop(0, NBLK, eblk, 0)
    plsc.subcore_barrier()
    pltpu.sync_copy(dacc_sh.at[pl.ds(base, RS)], deg_hbm.at[c, pl.ds(base, RS)])


_hist = pl.kernel(
    _hist_body,
    jax.ShapeDtypeStruct((NC, N2, DW), jnp.float32),
    mesh=_mesh,
    compiler_params=pltpu.CompilerParams(use_tc_tiling_on_sc=False),
    scratch_types=[
        pltpu.VMEM((NBLK, BLK_E), jnp.int32),
        pltpu.VMEM((BLK_E, DW), jnp.float32),
        pltpu.VMEM_SHARED((N2, DW), jnp.float32),
    ],
)


NBUF = 4
NGRP = NBLK // (2 * NBUF)


def _agg_body(g_hbm, src_hbm, dst_hbm, out_hbm, sidx_v, didx_v, msg_a, msg_b,
              acc_sh, gsem_a, ssem_a, gsem_b, ssem_b):
    c = lax.axis_index("c")
    s = lax.axis_index("s")
    wid = s * NC + c
    pltpu.sync_copy(src_hbm.at[wid], sidx_v)
    pltpu.sync_copy(dst_hbm.at[wid], didx_v)
    base = s * RS

    zeros16 = jnp.zeros((16,), jnp.float32)

    def zrow(r, carry):
        for q in range(DH // 16):
            msg_a[0, r, pl.ds(q * 16, 16)] = zeros16
        return carry

    lax.fori_loop(0, BLK_E, zrow, 0)
    for k in range(RS // BLK_E):
        pltpu.sync_copy(msg_a.at[0], acc_sh.at[pl.ds(base + k * BLK_E, BLK_E)])
    plsc.subcore_barrier()

    def _gather(j, buf_ref, sem):
        pltpu.async_copy(g_hbm.at[sidx_v.at[j]], buf_ref, sem)

    def _gather_wait(j, buf_ref, sem):
        pltpu.make_async_copy(g_hbm.at[sidx_v.at[j]], buf_ref, sem).wait()

    def _scat(j, buf_ref, sem):
        pltpu.async_copy(buf_ref, acc_sh.at[didx_v.at[j]], sem, add=True)

    def _scat_wait(j, buf_ref, sem):
        pltpu.make_async_copy(buf_ref, acc_sh.at[didx_v.at[j]], sem).wait()

    for b in range(NBUF):
        _gather(b, msg_a.at[b], gsem_a)

    def grp(gi, carry):
        ja = gi * 2 * NBUF
        jb = ja + NBUF
        for b in range(NBUF):
            _gather(jb + b, msg_b.at[b], gsem_b)
        for b in range(NBUF):
            _gather_wait(ja + b, msg_a.at[b], gsem_a)
        for b in range(NBUF):
            _scat(ja + b, msg_a.at[b], ssem_a)
        for b in range(NBUF):
            _scat_wait(ja + b, msg_a.at[b], ssem_a)

        @pl.when(gi < NGRP - 1)
        def _refill_a():
            for b in range(NBUF):
                _gather(ja + 2 * NBUF + b, msg_a.at[b], gsem_a)

        for b in range(NBUF):
            _gather_wait(jb + b, msg_b.at[b], gsem_b)
        for b in range(NBUF):
            _scat(jb + b, msg_b.at[b], ssem_b)
        for b in range(NBUF):
            _scat_wait(jb + b, msg_b.at[b], ssem_b)
        return carry

    lax.fori_loop(0, NGRP, grp, 0)
    plsc.subcore_barrier()
    pltpu.sync_copy(acc_sh.at[pl.ds(base, RS)], out_hbm.at[c].at[pl.ds(base, RS)])


_agg = pl.kernel(
    _agg_body,
    jax.ShapeDtypeStruct((NC, N2, DH), jnp.float32),
    mesh=_mesh,
    compiler_params=pltpu.CompilerParams(use_tc_tiling_on_sc=False),
    scratch_types=[
        pltpu.VMEM((NBLK, BLK_E), jnp.int32),
        pltpu.VMEM((NBLK, BLK_E), jnp.int32),
        pltpu.VMEM((NBUF, BLK_E, DH), jnp.float32),
        pltpu.VMEM((NBUF, BLK_E, DH), jnp.float32),
        pltpu.VMEM_SHARED((N2, DH), jnp.float32),
        pltpu.SemaphoreType.DMA,
        pltpu.SemaphoreType.DMA,
        pltpu.SemaphoreType.DMA,
        pltpu.SemaphoreType.DMA,
    ],
)


def _dinv_of(deg_blk):
    return lax.rsqrt(deg_blk[0, :, :1] + deg_blk[1, :, :1] + 1.0)


def _tc1_body(deg_ref, x_ref, w1_ref, g1_ref):
    dinv = _dinv_of(deg_ref[...])
    h = jnp.dot(x_ref[...], w1_ref[...], preferred_element_type=jnp.float32)
    g1_ref[...] = h * dinv


def _tc2_body(acc_ref, g1_ref, deg_ref, w2_ref, b1_ref, g2_ref):
    dinv = _dinv_of(deg_ref[...])
    a = acc_ref[0] + acc_ref[1] + g1_ref[...]
    h = jnp.maximum(a * dinv + b1_ref[...], 0.0)
    h2 = jnp.dot(h, w2_ref[...], preferred_element_type=jnp.float32)
    g2_ref[...] = h2 * dinv


def _tc3_body(acc_ref, g2_ref, deg_ref, wl_ref, b2_ref, bl_ref, y_ref):
    dinv = _dinv_of(deg_ref[...])
    a = acc_ref[0] + acc_ref[1] + g2_ref[...]
    h = jnp.maximum(a * dinv + b2_ref[...], 0.0)
    y_ref[...] = jnp.sum(h * wl_ref[...], axis=1) + bl_ref[0, 0]


_deg_spec = pl.BlockSpec((NC, BN, DW), lambda i: (0, i, 0))
_row_spec = pl.BlockSpec((BN, DH), lambda i: (i, 0))
_acc_spec = pl.BlockSpec((NC, BN, DH), lambda i: (0, i, 0))
_vec_spec = pl.BlockSpec((1, DH), lambda i: (0, 0))

_tc1 = pl.pallas_call(
    _tc1_body,
    grid=(N2 // BN,),
    in_specs=[
        _deg_spec,
        pl.BlockSpec((BN, D_IN), lambda i: (i, 0)),
        pl.BlockSpec((D_IN, DH), lambda i: (0, 0)),
    ],
    out_specs=_row_spec,
    out_shape=jax.ShapeDtypeStruct((N2, DH), jnp.float32),
)

_tc2 = pl.pallas_call(
    _tc2_body,
    grid=(N2 // BN,),
    in_specs=[
        _acc_spec,
        _row_spec,
        _deg_spec,
        pl.BlockSpec((DH, DH), lambda i: (0, 0)),
        _vec_spec,
    ],
    out_specs=_row_spec,
    out_shape=jax.ShapeDtypeStruct((N2, DH), jnp.float32),
)

_tc3 = pl.pallas_call(
    _tc3_body,
    grid=(N2 // BN,),
    in_specs=[
        _acc_spec,
        _row_spec,
        _deg_spec,
        _vec_spec,
        _vec_spec,
        _vec_spec,
    ],
    out_specs=pl.BlockSpec((BN,), lambda i: (i,)),
    out_shape=jax.ShapeDtypeStruct((N2,), jnp.float32),
)


def kernel(x, edge_index, W1, b1, W2, b2, Wl, bl):
    src = edge_index[0]
    dst = edge_index[1]
    pad = N + jnp.arange(E_PAD - E, dtype=edge_index.dtype) % (N2 - N)
    src_r = jnp.concatenate([src, pad]).reshape(NW, NBLK, BLK_E)
    dst_r = jnp.concatenate([dst, pad]).reshape(NW, NBLK, BLK_E)
    x_pad = jnp.pad(x, ((0, N2 - N), (0, 0)))

    deg_parts = _hist(dst_r)
    g1 = _tc1(deg_parts, x_pad, W1)
    acc1 = _agg(g1, src_r, dst_r)
    g2 = _tc2(acc1, g1, deg_parts, W2, b1.reshape(1, DH))
    acc2 = _agg(g2, src_r, dst_r)
    y = _tc3(
        acc2,
        g2,
        deg_parts,
        Wl.reshape(1, DH),
        b2.reshape(1, DH),
        jnp.broadcast_to(bl.reshape(1, 1), (1, DH)),
    )
    return y[:N]


# speedup vs baseline: 45.6314x; 1.0601x over previous
import functools

import jax
import jax.numpy as jnp
from jax import lax
from jax.experimental import pallas as pl
from jax.experimental.pallas import tpu as pltpu
from jax.experimental.pallas import tpu_sc as plsc

N = 10000
E = 320000
D_IN = 128
DH = 64

NC = 2
NS = 16
NW = NC * NS

N2 = 10240
RS = N2 // NS
EC = E // NW
BLK_E = 128
NBF = EC // BLK_E
TAIL = EC - NBF * BLK_E

NBUF = 3
NGRP = NBF // (2 * NBUF)

BN = 2048

_mesh = plsc.VectorSubcoreMesh(
    core_axis_name="c", subcore_axis_name="s", num_cores=NC, num_subcores=NS
)


DW = 16


def _hist_body(edge_hbm, deg_hbm, didx_v, ones_v, zbuf, dacc_sh):
    c = lax.axis_index("c")
    s = lax.axis_index("s")
    wid = s * NC + c
    pltpu.sync_copy(edge_hbm.at[1].at[pl.ds(wid * EC, EC)], didx_v)

    zeros16 = jnp.zeros((16,), jnp.float32)
    ones16 = jnp.ones((16,), jnp.float32)

    def fillrow(r, carry):
        zbuf[r, pl.ds(0, DW)] = zeros16
        ones_v[r, pl.ds(0, DW)] = ones16
        return carry

    lax.fori_loop(0, BLK_E, fillrow, 0)
    base = s * RS
    for k in range(RS // BLK_E):
        pltpu.sync_copy(zbuf, dacc_sh.at[pl.ds(base + k * BLK_E, BLK_E)])
    plsc.subcore_barrier()

    def eblk(j, carry):
        pltpu.sync_copy(
            ones_v, dacc_sh.at[didx_v.at[pl.ds(j * BLK_E, BLK_E)]], add=True
        )
        return carry

    lax.fori_loop(0, NBF, eblk, 0)
    pltpu.sync_copy(
        ones_v.at[pl.ds(0, TAIL)],
        dacc_sh.at[didx_v.at[pl.ds(NBF * BLK_E, TAIL)]],
        add=True,
    )
    plsc.subcore_barrier()
    pltpu.sync_copy(dacc_sh.at[pl.ds(base, RS)], deg_hbm.at[c, pl.ds(base, RS)])


_hist = pl.kernel(
    _hist_body,
    jax.ShapeDtypeStruct((NC, N2, DW), jnp.float32),
    mesh=_mesh,
    compiler_params=pltpu.CompilerParams(use_tc_tiling_on_sc=False),
    scratch_types=[
        pltpu.VMEM((EC,), jnp.int32),
        pltpu.VMEM((BLK_E, DW), jnp.float32),
        pltpu.VMEM((BLK_E, DW), jnp.float32),
        pltpu.VMEM_SHARED((N2, DW), jnp.float32),
    ],
)


def _agg_body(g_hbm, edge_hbm, out_hbm, sidx_v, didx_v, msg_a, msg_b, zbuf,
              acc_sh, gsem_a, ssem_a, gsem_b, ssem_b):
    c = lax.axis_index("c")
    s = lax.axis_index("s")
    wid = s * NC + c
    pltpu.sync_copy(edge_hbm.at[0].at[pl.ds(wid * EC, EC)], sidx_v)
    pltpu.sync_copy(edge_hbm.at[1].at[pl.ds(wid * EC, EC)], didx_v)
    base = s * RS

    zeros16 = jnp.zeros((16,), jnp.float32)

    def zrow(r, carry):
        for q in range(DH // 16):
            zbuf[r, pl.ds(q * 16, 16)] = zeros16
        return carry

    lax.fori_loop(0, BLK_E, zrow, 0)
    for k in range(RS // BLK_E):
        pltpu.sync_copy(zbuf, acc_sh.at[pl.ds(base + k * BLK_E, BLK_E)])
    plsc.subcore_barrier()

    def _src(j):
        return sidx_v.at[pl.ds(j * BLK_E, BLK_E)]

    def _dst(j):
        return didx_v.at[pl.ds(j * BLK_E, BLK_E)]

    def _gather(j, buf_ref, sem):
        pltpu.async_copy(g_hbm.at[_src(j)], buf_ref, sem)

    def _gather_wait(j, buf_ref, sem):
        pltpu.make_async_copy(g_hbm.at[_src(j)], buf_ref, sem).wait()

    def _scat(j, buf_ref, sem):
        pltpu.async_copy(buf_ref, acc_sh.at[_dst(j)], sem, add=True)

    def _scat_wait(j, buf_ref, sem):
        pltpu.make_async_copy(buf_ref, acc_sh.at[_dst(j)], sem).wait()

    for b in range(NBUF):
        _gather(b, msg_a.at[b], gsem_a)

    def grp(gi, carry):
        ja = gi * 2 * NBUF
        jb = ja + NBUF
        for b in range(NBUF):
            _gather(jb + b, msg_b.at[b], gsem_b)
        for b in range(NBUF):
            _gather_wait(ja + b, msg_a.at[b], gsem_a)
        for b in range(NBUF):
            _scat(ja + b, msg_a.at[b], ssem_a)
        for b in range(NBUF):
            _scat_wait(ja + b, msg_a.at[b], ssem_a)

        @pl.when(gi < NGRP - 1)
        def _refill_a():
            for b in range(NBUF):
                _gather(ja + 2 * NBUF + b, msg_a.at[b], gsem_a)

        for b in range(NBUF):
            _gather_wait(jb + b, msg_b.at[b], gsem_b)
        for b in range(NBUF):
            _scat(jb + b, msg_b.at[b], ssem_b)
        for b in range(NBUF):
            _scat_wait(jb + b, msg_b.at[b], ssem_b)
        return carry

    lax.fori_loop(0, NGRP, grp, 0)

    tail_src = sidx_v.at[pl.ds(NBF * BLK_E, TAIL)]
    tail_dst = didx_v.at[pl.ds(NBF * BLK_E, TAIL)]
    tbuf = msg_a.at[0].at[pl.ds(0, TAIL)]
    pltpu.sync_copy(g_hbm.at[tail_src], tbuf)
    pltpu.sync_copy(tbuf, acc_sh.at[tail_dst], add=True)

    plsc.subcore_barrier()
    pltpu.sync_copy(acc_sh.at[pl.ds(base, RS)], out_hbm.at[c, pl.ds(base, RS)])


_agg = pl.kernel(
    _agg_body,
    jax.ShapeDtypeStruct((NC, N2, DH), jnp.float32),
    mesh=_mesh,
    compiler_params=pltpu.CompilerParams(use_tc_tiling_on_sc=False),
    scratch_types=[
        pltpu.VMEM((EC,), jnp.int32),
        pltpu.VMEM((EC,), jnp.int32),
        pltpu.VMEM((NBUF, BLK_E, DH), jnp.float32),
        pltpu.VMEM((NBUF, BLK_E, DH), jnp.float32),
        pltpu.VMEM((BLK_E, DH), jnp.float32),
        pltpu.VMEM_SHARED((N2, DH), jnp.float32),
        pltpu.SemaphoreType.DMA,
        pltpu.SemaphoreType.DMA,
        pltpu.SemaphoreType.DMA,
        pltpu.SemaphoreType.DMA,
    ],
)


def _dinv_of(deg_blk):
    return lax.rsqrt(deg_blk[0, :, :1] + deg_blk[1, :, :1] + 1.0)


def _tc1_body(deg_ref, x_ref, w1_ref, g1_ref):
    dinv = _dinv_of(deg_ref[...])
    h = jnp.dot(x_ref[...], w1_ref[...], preferred_element_type=jnp.float32)
    g1_ref[...] = h * dinv


def _tc2_body(acc_ref, g1_ref, deg_ref, w2_ref, b1_ref, g2_ref):
    dinv = _dinv_of(deg_ref[...])
    a = acc_ref[0] + acc_ref[1] + g1_ref[...]
    h = jnp.maximum(a * dinv + b1_ref[...], 0.0)
    h2 = jnp.dot(h, w2_ref[...], preferred_element_type=jnp.float32)
    g2_ref[...] = h2 * dinv


def _tc3_body(acc_ref, g2_ref, deg_ref, wl_ref, b2_ref, bl_ref, y_ref):
    dinv = _dinv_of(deg_ref[...])
    a = acc_ref[0] + acc_ref[1] + g2_ref[...]
    h = jnp.maximum(a * dinv + b2_ref[...], 0.0)
    y_ref[...] = jnp.sum(h * wl_ref[...], axis=1) + bl_ref[0, 0]


_deg_spec = pl.BlockSpec((NC, BN, DW), lambda i: (0, i, 0))
_row_spec = pl.BlockSpec((BN, DH), lambda i: (i, 0))
_acc_spec = pl.BlockSpec((NC, BN, DH), lambda i: (0, i, 0))
_vec_spec = pl.BlockSpec((1, DH), lambda i: (0, 0))

_tc1 = pl.pallas_call(
    _tc1_body,
    grid=(N2 // BN,),
    in_specs=[
        _deg_spec,
        pl.BlockSpec((BN, D_IN), lambda i: (i, 0)),
        pl.BlockSpec((D_IN, DH), lambda i: (0, 0)),
    ],
    out_specs=_row_spec,
    out_shape=jax.ShapeDtypeStruct((N2, DH), jnp.float32),
)

_tc2 = pl.pallas_call(
    _tc2_body,
    grid=(N2 // BN,),
    in_specs=[
        _acc_spec,
        _row_spec,
        _deg_spec,
        pl.BlockSpec((DH, DH), lambda i: (0, 0)),
        _vec_spec,
    ],
    out_specs=_row_spec,
    out_shape=jax.ShapeDtypeStruct((N2, DH), jnp.float32),
)

_tc3 = pl.pallas_call(
    _tc3_body,
    grid=(N2 // BN,),
    in_specs=[
        _acc_spec,
        _row_spec,
        _deg_spec,
        _vec_spec,
        _vec_spec,
        _vec_spec,
    ],
    out_specs=pl.BlockSpec((BN,), lambda i: (i,)),
    out_shape=jax.ShapeDtypeStruct((N2,), jnp.float32),
)


def kernel(x, edge_index, W1, b1, W2, b2, Wl, bl):
    x_pad = jnp.pad(x, ((0, N2 - N), (0, 0)))

    deg_parts = _hist(edge_index)
    g1 = _tc1(deg_parts, x_pad, W1)
    acc1 = _agg(g1, edge_index)
    g2 = _tc2(acc1, g1, deg_parts, W2, b1.reshape(1, DH))
    acc2 = _agg(g2, edge_index)
    y = _tc3(
        acc2,
        g2,
        deg_parts,
        Wl.reshape(1, DH),
        b2.reshape(1, DH),
        jnp.broadcast_to(bl.reshape(1, 1), (1, DH)),
    )
    return y[:N]


# speedup vs baseline: 47.6166x; 1.0435x over previous
import functools

import jax
import jax.numpy as jnp
from jax import lax
from jax.experimental import pallas as pl
from jax.experimental.pallas import tpu as pltpu
from jax.experimental.pallas import tpu_sc as plsc

N = 10000
E = 320000
D_IN = 128
DH = 64

NC = 2
NS = 16
NW = NC * NS

N2 = 10240
RS = N2 // NS
EC = E // NW
BLK_E = 128
NBF = EC // BLK_E
TAIL = EC - NBF * BLK_E

NB = 7

BN = 2048

_mesh = plsc.VectorSubcoreMesh(
    core_axis_name="c", subcore_axis_name="s", num_cores=NC, num_subcores=NS
)


DW = 16


def _hist_body(edge_hbm, deg_hbm, didx_v, ones_v, zbuf, dacc_sh):
    c = lax.axis_index("c")
    s = lax.axis_index("s")
    wid = s * NC + c
    pltpu.sync_copy(edge_hbm.at[1].at[pl.ds(wid * EC, EC)], didx_v)

    zeros16 = jnp.zeros((16,), jnp.float32)
    ones16 = jnp.ones((16,), jnp.float32)

    def fillrow(r, carry):
        zbuf[r, pl.ds(0, DW)] = zeros16
        ones_v[r, pl.ds(0, DW)] = ones16
        return carry

    lax.fori_loop(0, BLK_E, fillrow, 0)
    base = s * RS
    for k in range(RS // BLK_E):
        pltpu.sync_copy(zbuf, dacc_sh.at[pl.ds(base + k * BLK_E, BLK_E)])
    plsc.subcore_barrier()

    def eblk(j, carry):
        pltpu.sync_copy(
            ones_v, dacc_sh.at[didx_v.at[pl.ds(j * BLK_E, BLK_E)]], add=True
        )
        return carry

    lax.fori_loop(0, NBF, eblk, 0)
    pltpu.sync_copy(
        ones_v.at[pl.ds(0, TAIL)],
        dacc_sh.at[didx_v.at[pl.ds(NBF * BLK_E, TAIL)]],
        add=True,
    )
    plsc.subcore_barrier()
    pltpu.sync_copy(dacc_sh.at[pl.ds(base, RS)], deg_hbm.at[c, pl.ds(base, RS)])


_hist = pl.kernel(
    _hist_body,
    jax.ShapeDtypeStruct((NC, N2, DW), jnp.float32),
    mesh=_mesh,
    compiler_params=pltpu.CompilerParams(use_tc_tiling_on_sc=False),
    scratch_types=[
        pltpu.VMEM((EC,), jnp.int32),
        pltpu.VMEM((BLK_E, DW), jnp.float32),
        pltpu.VMEM((BLK_E, DW), jnp.float32),
        pltpu.VMEM_SHARED((N2, DW), jnp.float32),
    ],
)


def _agg_body(g_hbm, edge_hbm, out_hbm, sidx_v, didx_v, msg, zbuf,
              acc_sh, *sems):
    c = lax.axis_index("c")
    s = lax.axis_index("s")
    wid = s * NC + c
    pltpu.sync_copy(edge_hbm.at[0].at[pl.ds(wid * EC, EC)], sidx_v)
    pltpu.sync_copy(edge_hbm.at[1].at[pl.ds(wid * EC, EC)], didx_v)
    base = s * RS

    zeros16 = jnp.zeros((16,), jnp.float32)

    def zrow(r, carry):
        for q in range(DH // 16):
            zbuf[r, pl.ds(q * 16, 16)] = zeros16
        return carry

    lax.fori_loop(0, BLK_E, zrow, 0)
    for k in range(RS // BLK_E):
        pltpu.sync_copy(zbuf, acc_sh.at[pl.ds(base + k * BLK_E, BLK_E)])
    plsc.subcore_barrier()

    def _src(j):
        return sidx_v.at[pl.ds(j * BLK_E, BLK_E)]

    def _dst(j):
        return didx_v.at[pl.ds(j * BLK_E, BLK_E)]

    def _gather(j, buf_ref, sem):
        pltpu.async_copy(g_hbm.at[_src(j)], buf_ref, sem)

    def _gather_wait(j, buf_ref, sem):
        pltpu.make_async_copy(g_hbm.at[_src(j)], buf_ref, sem).wait()

    def _scat(j, buf_ref, sem):
        pltpu.async_copy(buf_ref, acc_sh.at[_dst(j)], sem, add=True)

    def _scat_wait(j, buf_ref, sem):
        pltpu.make_async_copy(buf_ref, acc_sh.at[_dst(j)], sem).wait()

    for b in range(NB):
        _gather(b, msg.at[b], sems[b])
    for j in range(NBF):
        b = j % NB
        _gather_wait(j, msg.at[b], sems[b])
        _scat(j, msg.at[b], sems[b])
        _scat_wait(j, msg.at[b], sems[b])
        if j + NB < NBF:
            _gather(j + NB, msg.at[b], sems[b])

    tail_src = sidx_v.at[pl.ds(NBF * BLK_E, TAIL)]
    tail_dst = didx_v.at[pl.ds(NBF * BLK_E, TAIL)]
    tbuf = msg.at[0].at[pl.ds(0, TAIL)]
    pltpu.sync_copy(g_hbm.at[tail_src], tbuf)
    pltpu.sync_copy(tbuf, acc_sh.at[tail_dst], add=True)

    plsc.subcore_barrier()
    pltpu.sync_copy(acc_sh.at[pl.ds(base, RS)], out_hbm.at[c, pl.ds(base, RS)])


_agg = pl.kernel(
    _agg_body,
    jax.ShapeDtypeStruct((NC, N2, DH), jnp.float32),
    mesh=_mesh,
    compiler_params=pltpu.CompilerParams(use_tc_tiling_on_sc=False),
    scratch_types=[
        pltpu.VMEM((EC,), jnp.int32),
        pltpu.VMEM((EC,), jnp.int32),
        pltpu.VMEM((NB, BLK_E, DH), jnp.float32),
        pltpu.VMEM((BLK_E, DH), jnp.float32),
        pltpu.VMEM_SHARED((N2, DH), jnp.float32),
    ] + [pltpu.SemaphoreType.DMA] * NB,
)


def _dinv_of(deg_blk):
    return lax.rsqrt(deg_blk[0, :, :1] + deg_blk[1, :, :1] + 1.0)


def _tc1_body(deg_ref, x_ref, w1_ref, g1_ref):
    dinv = _dinv_of(deg_ref[...])
    h = jnp.dot(x_ref[...], w1_ref[...], preferred_element_type=jnp.float32)
    g1_ref[...] = h * dinv


def _tc2_body(acc_ref, g1_ref, deg_ref, w2_ref, b1_ref, g2_ref):
    dinv = _dinv_of(deg_ref[...])
    a = acc_ref[0] + acc_ref[1] + g1_ref[...]
    h = jnp.maximum(a * dinv + b1_ref[...], 0.0)
    h2 = jnp.dot(h, w2_ref[...], preferred_element_type=jnp.float32)
    g2_ref[...] = h2 * dinv


def _tc3_body(acc_ref, g2_ref, deg_ref, wl_ref, b2_ref, bl_ref, y_ref):
    dinv = _dinv_of(deg_ref[...])
    a = acc_ref[0] + acc_ref[1] + g2_ref[...]
    h = jnp.maximum(a * dinv + b2_ref[...], 0.0)
    y_ref[...] = jnp.sum(h * wl_ref[...], axis=1) + bl_ref[0, 0]


_deg_spec = pl.BlockSpec((NC, BN, DW), lambda i: (0, i, 0))
_row_spec = pl.BlockSpec((BN, DH), lambda i: (i, 0))
_acc_spec = pl.BlockSpec((NC, BN, DH), lambda i: (0, i, 0))
_vec_spec = pl.BlockSpec((1, DH), lambda i: (0, 0))

_tc1 = pl.pallas_call(
    _tc1_body,
    grid=(N2 // BN,),
    in_specs=[
        _deg_spec,
        pl.BlockSpec((BN, D_IN), lambda i: (i, 0)),
        pl.BlockSpec((D_IN, DH), lambda i: (0, 0)),
    ],
    out_specs=_row_spec,
    out_shape=jax.ShapeDtypeStruct((N2, DH), jnp.float32),
)

_tc2 = pl.pallas_call(
    _tc2_body,
    grid=(N2 // BN,),
    in_specs=[
        _acc_spec,
        _row_spec,
        _deg_spec,
        pl.BlockSpec((DH, DH), lambda i: (0, 0)),
        _vec_spec,
    ],
    out_specs=_row_spec,
    out_shape=jax.ShapeDtypeStruct((N2, DH), jnp.float32),
)

_tc3 = pl.pallas_call(
    _tc3_body,
    grid=(N2 // BN,),
    in_specs=[
        _acc_spec,
        _row_spec,
        _deg_spec,
        _vec_spec,
        _vec_spec,
        _vec_spec,
    ],
    out_specs=pl.BlockSpec((BN,), lambda i: (i,)),
    out_shape=jax.ShapeDtypeStruct((N2,), jnp.float32),
)


def kernel(x, edge_index, W1, b1, W2, b2, Wl, bl):
    x_pad = jnp.pad(x, ((0, N2 - N), (0, 0)))

    deg_parts = _hist(edge_index)
    g1 = _tc1(deg_parts, x_pad, W1)
    acc1 = _agg(g1, edge_index)
    g2 = _tc2(acc1, g1, deg_parts, W2, b1.reshape(1, DH))
    acc2 = _agg(g2, edge_index)
    y = _tc3(
        acc2,
        g2,
        deg_parts,
        Wl.reshape(1, DH),
        b2.reshape(1, DH),
        jnp.broadcast_to(bl.reshape(1, 1), (1, DH)),
    )
    return y[:N]


# speedup vs baseline: 47.8762x; 1.0055x over previous
import functools

import jax
import jax.numpy as jnp
from jax import lax
from jax.experimental import pallas as pl
from jax.experimental.pallas import tpu as pltpu
from jax.experimental.pallas import tpu_sc as plsc

N = 10000
E = 320000
D_IN = 128
DH = 64

NC = 2
NS = 16
NW = NC * NS

N2 = 10240
RS = N2 // NS
EC = E // NW
BLK_E = 128
NBF = EC // BLK_E
TAIL = EC - NBF * BLK_E

NB = 7

BN = 5120

_mesh = plsc.VectorSubcoreMesh(
    core_axis_name="c", subcore_axis_name="s", num_cores=NC, num_subcores=NS
)


DW = 16


def _hist_body(edge_hbm, deg_hbm, didx_v, ones_v, zbuf, dacc_sh):
    c = lax.axis_index("c")
    s = lax.axis_index("s")
    wid = s * NC + c
    pltpu.sync_copy(edge_hbm.at[1].at[pl.ds(wid * EC, EC)], didx_v)

    zeros16 = jnp.zeros((16,), jnp.float32)
    ones16 = jnp.ones((16,), jnp.float32)

    def fillrow(r, carry):
        zbuf[r, pl.ds(0, DW)] = zeros16
        ones_v[r, pl.ds(0, DW)] = ones16
        return carry

    lax.fori_loop(0, BLK_E, fillrow, 0)
    base = s * RS
    for k in range(RS // BLK_E):
        pltpu.sync_copy(zbuf, dacc_sh.at[pl.ds(base + k * BLK_E, BLK_E)])
    plsc.subcore_barrier()

    def eblk(j, carry):
        pltpu.sync_copy(
            ones_v, dacc_sh.at[didx_v.at[pl.ds(j * BLK_E, BLK_E)]], add=True
        )
        return carry

    lax.fori_loop(0, NBF, eblk, 0)
    pltpu.sync_copy(
        ones_v.at[pl.ds(0, TAIL)],
        dacc_sh.at[didx_v.at[pl.ds(NBF * BLK_E, TAIL)]],
        add=True,
    )
    plsc.subcore_barrier()
    pltpu.sync_copy(dacc_sh.at[pl.ds(base, RS)], deg_hbm.at[c, pl.ds(base, RS)])


_hist = pl.kernel(
    _hist_body,
    jax.ShapeDtypeStruct((NC, N2, DW), jnp.float32),
    mesh=_mesh,
    compiler_params=pltpu.CompilerParams(use_tc_tiling_on_sc=False),
    scratch_types=[
        pltpu.VMEM((EC,), jnp.int32),
        pltpu.VMEM((BLK_E, DW), jnp.float32),
        pltpu.VMEM((BLK_E, DW), jnp.float32),
        pltpu.VMEM_SHARED((N2, DW), jnp.float32),
    ],
)


def _agg_body(g_hbm, edge_hbm, out_hbm, sidx_v, didx_v, msg, zbuf,
              acc_sh, *sems):
    c = lax.axis_index("c")
    s = lax.axis_index("s")
    wid = s * NC + c
    pltpu.sync_copy(edge_hbm.at[0].at[pl.ds(wid * EC, EC)], sidx_v)
    pltpu.sync_copy(edge_hbm.at[1].at[pl.ds(wid * EC, EC)], didx_v)
    base = s * RS

    zeros16 = jnp.zeros((16,), jnp.float32)

    def zrow(r, carry):
        for q in range(DH // 16):
            zbuf[r, pl.ds(q * 16, 16)] = zeros16
        return carry

    lax.fori_loop(0, BLK_E, zrow, 0)
    for k in range(RS // BLK_E):
        pltpu.sync_copy(zbuf, acc_sh.at[pl.ds(base + k * BLK_E, BLK_E)])
    plsc.subcore_barrier()

    def _src(j):
        return sidx_v.at[pl.ds(j * BLK_E, BLK_E)]

    def _dst(j):
        return didx_v.at[pl.ds(j * BLK_E, BLK_E)]

    def _gather(j, buf_ref, sem):
        pltpu.async_copy(g_hbm.at[_src(j)], buf_ref, sem)

    def _gather_wait(j, buf_ref, sem):
        pltpu.make_async_copy(g_hbm.at[_src(j)], buf_ref, sem).wait()

    def _scat(j, buf_ref, sem):
        pltpu.async_copy(buf_ref, acc_sh.at[_dst(j)], sem, add=True)

    def _scat_wait(j, buf_ref, sem):
        pltpu.make_async_copy(buf_ref, acc_sh.at[_dst(j)], sem).wait()

    for b in range(NB):
        _gather(b, msg.at[b], sems[b])
    for j in range(NBF):
        b = j % NB
        _gather_wait(j, msg.at[b], sems[b])
        _scat(j, msg.at[b], sems[b])
        _scat_wait(j, msg.at[b], sems[b])
        if j + NB < NBF:
            _gather(j + NB, msg.at[b], sems[b])

    tail_src = sidx_v.at[pl.ds(NBF * BLK_E, TAIL)]
    tail_dst = didx_v.at[pl.ds(NBF * BLK_E, TAIL)]
    tbuf = msg.at[0].at[pl.ds(0, TAIL)]
    pltpu.sync_copy(g_hbm.at[tail_src], tbuf)
    pltpu.sync_copy(tbuf, acc_sh.at[tail_dst], add=True)

    plsc.subcore_barrier()
    pltpu.sync_copy(acc_sh.at[pl.ds(base, RS)], out_hbm.at[c, pl.ds(base, RS)])


_agg = pl.kernel(
    _agg_body,
    jax.ShapeDtypeStruct((NC, N2, DH), jnp.float32),
    mesh=_mesh,
    compiler_params=pltpu.CompilerParams(use_tc_tiling_on_sc=False),
    scratch_types=[
        pltpu.VMEM((EC,), jnp.int32),
        pltpu.VMEM((EC,), jnp.int32),
        pltpu.VMEM((NB, BLK_E, DH), jnp.float32),
        pltpu.VMEM((BLK_E, DH), jnp.float32),
        pltpu.VMEM_SHARED((N2, DH), jnp.float32),
    ] + [pltpu.SemaphoreType.DMA] * NB,
)


def _dinv_of(deg_blk):
    return lax.rsqrt(deg_blk[0, :, :1] + deg_blk[1, :, :1] + 1.0)


def _tc1_body(deg_ref, x_ref, w1_ref, g1_ref):
    dinv = _dinv_of(deg_ref[...])
    h = jnp.dot(x_ref[...], w1_ref[...], preferred_element_type=jnp.float32)
    g1_ref[...] = h * dinv


def _tc2_body(acc_ref, g1_ref, deg_ref, w2_ref, b1_ref, g2_ref):
    dinv = _dinv_of(deg_ref[...])
    a = acc_ref[0] + acc_ref[1] + g1_ref[...]
    h = jnp.maximum(a * dinv + b1_ref[...], 0.0)
    h2 = jnp.dot(h, w2_ref[...], preferred_element_type=jnp.float32)
    g2_ref[...] = h2 * dinv


def _tc3_body(acc_ref, g2_ref, deg_ref, wl_ref, b2_ref, bl_ref, y_ref):
    dinv = _dinv_of(deg_ref[...])
    a = acc_ref[0] + acc_ref[1] + g2_ref[...]
    h = jnp.maximum(a * dinv + b2_ref[...], 0.0)
    y_ref[...] = jnp.sum(h * wl_ref[...], axis=1) + bl_ref[0, 0]


_deg_spec = pl.BlockSpec((NC, BN, DW), lambda i: (0, i, 0))
_row_spec = pl.BlockSpec((BN, DH), lambda i: (i, 0))
_acc_spec = pl.BlockSpec((NC, BN, DH), lambda i: (0, i, 0))
_vec_spec = pl.BlockSpec((1, DH), lambda i: (0, 0))

_tc1 = pl.pallas_call(
    _tc1_body,
    grid=(N2 // BN,),
    in_specs=[
        _deg_spec,
        pl.BlockSpec((BN, D_IN), lambda i: (i, 0)),
        pl.BlockSpec((D_IN, DH), lambda i: (0, 0)),
    ],
    out_specs=_row_spec,
    out_shape=jax.ShapeDtypeStruct((N2, DH), jnp.float32),
)

_tc2 = pl.pallas_call(
    _tc2_body,
    grid=(N2 // BN,),
    in_specs=[
        _acc_spec,
        _row_spec,
        _deg_spec,
        pl.BlockSpec((DH, DH), lambda i: (0, 0)),
        _vec_spec,
    ],
    out_specs=_row_spec,
    out_shape=jax.ShapeDtypeStruct((N2, DH), jnp.float32),
)

_tc3 = pl.pallas_call(
    _tc3_body,
    grid=(N2 // BN,),
    in_specs=[
        _acc_spec,
        _row_spec,
        _deg_spec,
        _vec_spec,
        _vec_spec,
        _vec_spec,
    ],
    out_specs=pl.BlockSpec((BN,), lambda i: (i,)),
    out_shape=jax.ShapeDtypeStruct((N2,), jnp.float32),
)


def kernel(x, edge_index, W1, b1, W2, b2, Wl, bl):
    x_pad = jnp.pad(x, ((0, N2 - N), (0, 0)))

    deg_parts = _hist(edge_index)
    g1 = _tc1(deg_parts, x_pad, W1)
    acc1 = _agg(g1, edge_index)
    g2 = _tc2(acc1, g1, deg_parts, W2, b1.reshape(1, DH))
    acc2 = _agg(g2, edge_index)
    y = _tc3(
        acc2,
        g2,
        deg_parts,
        Wl.reshape(1, DH),
        b2.reshape(1, DH),
        jnp.broadcast_to(bl.reshape(1, 1), (1, DH)),
    )
    return y[:N]
